# Initial kernel scaffold; baseline (speedup 1.0000x reference)
import jax
import jax.numpy as jnp
from jax.experimental import pallas as pl


def kernel(hidden_states, ln1_w, ln2_w, Wqkv, bqkv, Wproj, bproj, Wgate, Wup, Wdown):
    raise NotImplementedError("write your pallas kernel here")


# speedup vs baseline: 2.1332x; 2.1332x over previous
import jax
import jax.numpy as jnp
from jax.experimental import pallas as pl
from jax.experimental.pallas import tpu as pltpu

DIM = 1024
INNER = 4096
HEADS = 8
HD = DIM // HEADS
EPS = 1e-5

BM_QKV = 512
BM_ATTN = 1024
BM_PROJ = 1024
BM_MLP = 1024
CN_MLP = 512


def _qkv_krn(x_ref, w_ref, b_ref, g_ref, o_ref):
    x = x_ref[...]
    var = jnp.mean(x * x, axis=-1, keepdims=True)
    h = x * jax.lax.rsqrt(var + EPS) * g_ref[...]
    o_ref[...] = (
        jnp.dot(h, w_ref[...], preferred_element_type=jnp.float32) + b_ref[...]
    )


def _attn_krn(q_ref, k_ref, v_ref, o_ref):
    q = q_ref[0]
    k = k_ref[0]
    v = v_ref[0]
    logits = jax.lax.dot_general(
        q, k, (((1,), (1,)), ((), ())), preferred_element_type=jnp.float32
    ) * (HD ** -0.5)
    m = jnp.max(logits, axis=-1, keepdims=True)
    e = jnp.exp(logits - m)
    s = jnp.sum(e, axis=-1, keepdims=True)
    o_ref[0] = jnp.dot(e / s, v, preferred_element_type=jnp.float32)


def _proj_krn(o_ref, w_ref, b_ref, res_ref, g_ref, y_ref, h2_ref):
    y = (
        jnp.dot(o_ref[...], w_ref[...], preferred_element_type=jnp.float32)
        + b_ref[...]
        + res_ref[...]
    )
    y_ref[...] = y
    var = jnp.mean(y * y, axis=-1, keepdims=True)
    h2_ref[...] = y * jax.lax.rsqrt(var + EPS) * g_ref[...]


def _mlp_krn(x_ref, wg_ref, wu_ref, wd_ref, y_ref, o_ref):
    j = pl.program_id(1)
    x = x_ref[...]
    g = jnp.dot(x, wg_ref[...], preferred_element_type=jnp.float32)
    u = jnp.dot(x, wu_ref[...], preferred_element_type=jnp.float32)
    a = (0.5 * g * (1.0 + jax.lax.erf(g * (2.0 ** -0.5)))) * u
    contrib = jnp.dot(a, wd_ref[...], preferred_element_type=jnp.float32)

    @pl.when(j == 0)
    def _():
        o_ref[...] = y_ref[...] + contrib

    @pl.when(j > 0)
    def _():
        o_ref[...] += contrib


def kernel(hidden_states, ln1_w, ln2_w, Wqkv, bqkv, Wproj, bproj, Wgate, Wup, Wdown):
    B, L, D = hidden_states.shape
    R = B * L
    x2 = hidden_states.reshape(R, D)

    qkv = pl.pallas_call(
        _qkv_krn,
        grid=(R // BM_QKV,),
        in_specs=[
            pl.BlockSpec((BM_QKV, D), lambda i: (i, 0)),
            pl.BlockSpec((D, 3 * D), lambda i: (0, 0)),
            pl.BlockSpec((1, 3 * D), lambda i: (0, 0)),
            pl.BlockSpec((1, D), lambda i: (0, 0)),
        ],
        out_specs=pl.BlockSpec((BM_QKV, 3 * D), lambda i: (i, 0)),
        out_shape=jax.ShapeDtypeStruct((R, 3 * D), jnp.float32),
        compiler_params=pltpu.CompilerParams(
            dimension_semantics=("parallel",),
        ),
    )(x2, Wqkv, bqkv.reshape(1, 3 * D), ln1_w.reshape(1, D))

    qkv3 = qkv.reshape(B, L, 3 * D)
    attn_out = pl.pallas_call(
        _attn_krn,
        grid=(B, HEADS, L // BM_ATTN),
        in_specs=[
            pl.BlockSpec((1, BM_ATTN, HD), lambda b, h, i: (b, i, h)),
            pl.BlockSpec((1, L, HD), lambda b, h, i: (b, 0, HEADS + h)),
            pl.BlockSpec((1, L, HD), lambda b, h, i: (b, 0, 2 * HEADS + h)),
        ],
        out_specs=pl.BlockSpec((1, BM_ATTN, HD), lambda b, h, i: (b, i, h)),
        out_shape=jax.ShapeDtypeStruct((B, L, D), jnp.float32),
        compiler_params=pltpu.CompilerParams(
            dimension_semantics=("parallel", "parallel", "parallel"),
        ),
    )(qkv3, qkv3, qkv3)

    o2 = attn_out.reshape(R, D)
    y, h2 = pl.pallas_call(
        _proj_krn,
        grid=(R // BM_PROJ,),
        in_specs=[
            pl.BlockSpec((BM_PROJ, D), lambda i: (i, 0)),
            pl.BlockSpec((D, D), lambda i: (0, 0)),
            pl.BlockSpec((1, D), lambda i: (0, 0)),
            pl.BlockSpec((BM_PROJ, D), lambda i: (i, 0)),
            pl.BlockSpec((1, D), lambda i: (0, 0)),
        ],
        out_specs=[
            pl.BlockSpec((BM_PROJ, D), lambda i: (i, 0)),
            pl.BlockSpec((BM_PROJ, D), lambda i: (i, 0)),
        ],
        out_shape=[
            jax.ShapeDtypeStruct((R, D), jnp.float32),
            jax.ShapeDtypeStruct((R, D), jnp.float32),
        ],
        compiler_params=pltpu.CompilerParams(
            dimension_semantics=("parallel",),
        ),
    )(o2, Wproj, bproj.reshape(1, D), x2, ln2_w.reshape(1, D))

    out = pl.pallas_call(
        _mlp_krn,
        grid=(R // BM_MLP, INNER // CN_MLP),
        in_specs=[
            pl.BlockSpec((BM_MLP, D), lambda i, j: (i, 0)),
            pl.BlockSpec((D, CN_MLP), lambda i, j: (0, j)),
            pl.BlockSpec((D, CN_MLP), lambda i, j: (0, j)),
            pl.BlockSpec((CN_MLP, D), lambda i, j: (j, 0)),
            pl.BlockSpec((BM_MLP, D), lambda i, j: (i, 0)),
        ],
        out_specs=pl.BlockSpec((BM_MLP, D), lambda i, j: (i, 0)),
        out_shape=jax.ShapeDtypeStruct((R, D), jnp.float32),
        compiler_params=pltpu.CompilerParams(
            dimension_semantics=("parallel", "arbitrary"),
        ),
    )(h2, Wgate, Wup, Wdown, y)

    return out.reshape(B, L, D)
